# baseline (speedup 1.0000x reference)
import jax
import jax.numpy as jnp
from jax.experimental import pallas as pl
from jax.experimental.pallas import tpu as pltpu

_IN = 256
_OUT = 256
_NC = 6553
_CH = 128
_NCH = -(-_NC // _CH)

_BM = 8192
_NSTEP = 16 * 16384 // _BM
_K = 4


def _body(x_ref, w_ref, b_ref, fk_ref, vals_ref, o_ref, weff_ref,
          ring_ref, in_sems, obuf_ref, out_sems):
    j = pl.program_id(0)

    def start_load(step):
        slot = jax.lax.rem(step, _K)
        pltpu.make_async_copy(
            x_ref.at[pl.ds(pl.multiple_of(step * _BM, _BM), _BM), :],
            ring_ref.at[slot], in_sems.at[slot]).start()

    @pl.when(j == 0)
    def _prefetch():
        for s in range(_K - 1):
            start_load(s)

    @pl.when(j == 0)
    def _build():
        iota_o = jax.lax.broadcasted_iota(jnp.int32, (_OUT, _CH), 0)
        a_idx = jax.lax.broadcasted_iota(jnp.int32, (_CH, _CH), 0)
        b_idx = jax.lax.broadcasted_iota(jnp.int32, (_CH, _CH), 1)
        tri = jnp.where(a_idx > b_idx, 1.0, 0.0)

        weff_ref[...] = jnp.zeros((_OUT, _IN), jnp.float32)

        def chunk_delta(t):
            fk = fk_ref[t]
            vals = vals_ref[t]
            r = jax.lax.shift_right_logical(fk, 8)
            c = jnp.bitwise_and(fk, 255)
            ohr = jnp.where(iota_o == r, 1.0, 0.0)
            eqc = iota_o == c
            ohc = jnp.where(eqc, 1.0, 0.0)
            fkb = jnp.broadcast_to(fk, (_CH, _CH))
            same = fkb.T == fkb
            later_dup = jnp.sum(jnp.where(same, tri, 0.0), axis=0,
                                keepdims=True)
            keep = jnp.where(later_dup == 0.0, 1.0, 0.0)
            ohcv = jnp.where(eqc, 0.1 * vals * keep, 0.0)
            rhs = jnp.concatenate([ohcv, ohc], axis=0)
            both = jax.lax.dot_general(ohr, rhs, (((1,), (1,)), ((), ())),
                                       preferred_element_type=jnp.float32)
            return both[:, :_IN], both[:, _IN:]

        def chunk2(i, carry):
            d0, t0 = chunk_delta(2 * i)
            d1, t1 = chunk_delta(2 * i + 1)
            cur = weff_ref[...]
            weff_ref[...] = jnp.where(t1 > 0.5, d1,
                                      jnp.where(t0 > 0.5, d0, cur))
            return carry

        jax.lax.fori_loop(0, _NCH // 2, chunk2, 0)
        weff_ref[...] = weff_ref[...] + w_ref[...]

    @pl.when(j + (_K - 1) < _NSTEP)
    def _refill():
        start_load(j + _K - 1)

    slot = jax.lax.rem(j, _K)
    pltpu.make_async_copy(
        x_ref.at[pl.ds(0, _BM), :], ring_ref.at[slot],
        in_sems.at[slot]).wait()
    acc = jax.lax.dot_general(
        ring_ref[slot], weff_ref[...], (((1,), (1,)), ((), ())),
        preferred_element_type=jnp.float32)

    oslot = jax.lax.rem(j, 2)

    def out_wait(s):
        pltpu.make_async_copy(
            obuf_ref.at[s], o_ref.at[pl.ds(0, _BM), :],
            out_sems.at[s]).wait()

    @pl.when(j >= 2)
    def _drain_prev():
        out_wait(oslot)

    obuf_ref[oslot] = acc + b_ref[...]
    pltpu.make_async_copy(
        obuf_ref.at[oslot],
        o_ref.at[pl.ds(pl.multiple_of(j * _BM, _BM), _BM), :],
        out_sems.at[oslot]).start()

    @pl.when(j == _NSTEP - 1)
    def _drain_all():
        out_wait(1 - oslot)
        out_wait(oslot)


def kernel(x, W, b, row_indices, col_indices, values):
    bsz, seq, d = x.shape
    rows_total = bsz * seq
    x2 = x.reshape(rows_total, d)
    b2 = b.reshape(1, _OUT)

    fk = row_indices * _IN + col_indices
    pad = _NCH * _CH - _NC
    fk = jnp.concatenate([fk, jnp.broadcast_to(fk[-1:], (pad,))])
    vp = jnp.concatenate([values, jnp.broadcast_to(values[-1:], (pad,))])
    fk3 = fk.reshape(_NCH, 1, _CH)
    vals3 = vp.reshape(_NCH, 1, _CH).astype(jnp.float32)

    nstep = rows_total // _BM

    out2 = pl.pallas_call(
        _body,
        out_shape=jax.ShapeDtypeStruct((rows_total, _OUT), x.dtype),
        grid=(nstep,),
        in_specs=[
            pl.BlockSpec(memory_space=pl.ANY),
            pl.BlockSpec((_OUT, _IN), lambda j: (0, 0)),
            pl.BlockSpec((1, _OUT), lambda j: (0, 0)),
            pl.BlockSpec((_NCH, 1, _CH), lambda j: (0, 0, 0)),
            pl.BlockSpec((_NCH, 1, _CH), lambda j: (0, 0, 0)),
        ],
        out_specs=pl.BlockSpec(memory_space=pl.ANY),
        scratch_shapes=[pltpu.VMEM((_OUT, _IN), jnp.float32),
                        pltpu.VMEM((_K, _BM, _IN), jnp.float32),
                        pltpu.SemaphoreType.DMA((_K,)),
                        pltpu.VMEM((2, _BM, _OUT), jnp.float32),
                        pltpu.SemaphoreType.DMA((2,))],
        compiler_params=pltpu.CompilerParams(
            dimension_semantics=("arbitrary",),
            vmem_limit_bytes=58 * 1024 * 1024,
        ),
        name="small_world_layer",
    )(x2, W, b2, fk3, vals3)
    return out2.reshape(bsz, seq, _OUT)


# speedup vs baseline: 1.0145x; 1.0145x over previous
import jax
import jax.numpy as jnp
from jax.experimental import pallas as pl
from jax.experimental.pallas import tpu as pltpu

_IN = 256
_OUT = 256
_NC = 6553
_CH = 128
_NCH = -(-_NC // _CH)

_BM = 8192
_NSTEP = 16 * 16384 // _BM
_K = 5


def _body(x_ref, w_ref, b_ref, fk_ref, vals_ref, o_ref, weff_ref,
          ring_ref, in_sems):
    j = pl.program_id(0)

    def start_load(step):
        slot = jax.lax.rem(step, _K)
        pltpu.make_async_copy(
            x_ref.at[pl.ds(pl.multiple_of(step * _BM, _BM), _BM), :],
            ring_ref.at[slot], in_sems.at[slot]).start()

    @pl.when(j == 0)
    def _prefetch():
        for s in range(_K - 1):
            start_load(s)

    @pl.when(j == 0)
    def _build():
        iota_o = jax.lax.broadcasted_iota(jnp.int32, (_OUT, _CH), 0)
        a_idx = jax.lax.broadcasted_iota(jnp.int32, (_CH, _CH), 0)
        b_idx = jax.lax.broadcasted_iota(jnp.int32, (_CH, _CH), 1)
        tri = jnp.where(a_idx > b_idx, 1.0, 0.0)

        weff_ref[...] = jnp.zeros((_OUT, _IN), jnp.float32)

        def chunk_delta(t):
            fk = fk_ref[t]
            vals = vals_ref[t]
            r = jax.lax.shift_right_logical(fk, 8)
            c = jnp.bitwise_and(fk, 255)
            ohr = jnp.where(iota_o == r, 1.0, 0.0)
            eqc = iota_o == c
            ohc = jnp.where(eqc, 1.0, 0.0)
            fkb = jnp.broadcast_to(fk, (_CH, _CH))
            same = fkb.T == fkb
            later_dup = jnp.sum(jnp.where(same, tri, 0.0), axis=0,
                                keepdims=True)
            keep = jnp.where(later_dup == 0.0, 1.0, 0.0)
            ohcv = jnp.where(eqc, 0.1 * vals * keep, 0.0)
            rhs = jnp.concatenate([ohcv, ohc], axis=0)
            both = jax.lax.dot_general(ohr, rhs, (((1,), (1,)), ((), ())),
                                       preferred_element_type=jnp.float32)
            return both[:, :_IN], both[:, _IN:]

        def chunk2(i, carry):
            d0, t0 = chunk_delta(2 * i)
            d1, t1 = chunk_delta(2 * i + 1)
            cur = weff_ref[...]
            weff_ref[...] = jnp.where(t1 > 0.5, d1,
                                      jnp.where(t0 > 0.5, d0, cur))
            return carry

        jax.lax.fori_loop(0, _NCH // 2, chunk2, 0)
        weff_ref[...] = weff_ref[...] + w_ref[...]

    @pl.when(j + (_K - 1) < _NSTEP)
    def _refill():
        start_load(j + _K - 1)

    slot = jax.lax.rem(j, _K)
    pltpu.make_async_copy(
        x_ref.at[pl.ds(0, _BM), :], ring_ref.at[slot],
        in_sems.at[slot]).wait()
    acc = jax.lax.dot_general(
        ring_ref[slot], weff_ref[...], (((1,), (1,)), ((), ())),
        preferred_element_type=jnp.float32)
    o_ref[...] = acc + b_ref[...]


def kernel(x, W, b, row_indices, col_indices, values):
    bsz, seq, d = x.shape
    rows_total = bsz * seq
    x2 = x.reshape(rows_total, d)
    b2 = b.reshape(1, _OUT)

    fk = row_indices * _IN + col_indices
    pad = _NCH * _CH - _NC
    fk = jnp.concatenate([fk, jnp.broadcast_to(fk[-1:], (pad,))])
    vp = jnp.concatenate([values, jnp.broadcast_to(values[-1:], (pad,))])
    fk3 = fk.reshape(_NCH, 1, _CH)
    vals3 = vp.reshape(_NCH, 1, _CH).astype(jnp.float32)

    nstep = rows_total // _BM

    out2 = pl.pallas_call(
        _body,
        out_shape=jax.ShapeDtypeStruct((rows_total, _OUT), x.dtype),
        grid=(nstep,),
        in_specs=[
            pl.BlockSpec(memory_space=pl.ANY),
            pl.BlockSpec((_OUT, _IN), lambda j: (0, 0)),
            pl.BlockSpec((1, _OUT), lambda j: (0, 0)),
            pl.BlockSpec((_NCH, 1, _CH), lambda j: (0, 0, 0)),
            pl.BlockSpec((_NCH, 1, _CH), lambda j: (0, 0, 0)),
        ],
        out_specs=pl.BlockSpec((_BM, _OUT), lambda j: (j, 0)),
        scratch_shapes=[pltpu.VMEM((_OUT, _IN), jnp.float32),
                        pltpu.VMEM((_K, _BM, _IN), jnp.float32),
                        pltpu.SemaphoreType.DMA((_K,))],
        compiler_params=pltpu.CompilerParams(
            dimension_semantics=("arbitrary",),
            vmem_limit_bytes=58 * 1024 * 1024,
        ),
        name="small_world_layer",
    )(x2, W, b2, fk3, vals3)
    return out2.reshape(bsz, seq, _OUT)
